# baseline (speedup 1.0000x reference)
import functools

import jax
import jax.numpy as jnp
from jax import lax
from jax.experimental import pallas as pl
from jax.experimental.pallas import tpu as pltpu
from jax.experimental.pallas import tpu_sc as plsc

_OFF1 = 50000
_OFF2 = 80000
_BASE_DIM = 64
_B = 16384

_NC = 2
_NS = 16
_L = 16
_NW = _NC * _NS
_BPW = _B // _NW
_ISZ = 128
_NSTEP = _BPW // _ISZ

_R0 = 25088
_R1 = 7680
_R2 = 2560


def _sc_body(ids_hbm, t0, t1, t2, ep_hbm,
             ids_v, g0c, g1c, g2c, p0c, p1c, p2c, c0c, c1c, c2c,
             bufa, bufb, ep_all, sem_a, sem_b):
    sid = lax.axis_index("s")
    wid = sid * _NC + lax.axis_index("c")
    base = wid * _BPW
    pltpu.sync_copy(ids_hbm.at[pl.ds(base, _BPW)], ids_v)
    lane = lax.iota(jnp.int32, _L)
    trash = jnp.full((_L,), _BPW, jnp.int32)
    zero = jnp.zeros((_L,), jnp.int32)
    for c in range(_BPW // _L):
        pre = lane + c * _L
        sl = pl.ds(c * _L, _L)
        g0c[sl] = pre
        g1c[sl] = pre
        g2c[sl] = pre
        p0c[sl] = trash
        p1c[sl] = trash
        p2c[sl] = trash
        c0c[sl] = zero
        c1c[sl] = zero
        c2c[sl] = zero
    n0 = 0
    n1 = 0
    n2 = 0
    for c in range(_BPW // _L):
        v = ids_v[pl.ds(c * _L, _L)]
        pos = lane + c * _L
        in0 = v < _OFF1
        in1 = (v >= _OFF1) & (v < _OFF2)
        in2 = v >= _OFF2
        l0 = v + 1
        l1 = v - (_OFF1 - 1)
        l2 = v - (_OFF2 - 1)
        q0 = (l0 >= _R0).astype(jnp.int32)
        q1 = ((l1 >= _R1).astype(jnp.int32) +
              (l1 >= 2 * _R1).astype(jnp.int32) +
              (l1 >= 3 * _R1).astype(jnp.int32))
        q2 = (l2 >= _R2).astype(jnp.int32)
        for k in range(2, 8):
            q2 = q2 + (l2 >= k * _R2).astype(jnp.int32)
        s0 = plsc.cumsum(in0.astype(jnp.int32))
        s1 = plsc.cumsum(in1.astype(jnp.int32))
        s2 = plsc.cumsum(in2.astype(jnp.int32))
        d0 = jnp.where(in0, n0 + s0 - 1, _BPW + _L - 1)
        d1 = jnp.where(in1, n1 + s1 - 1, _BPW + _L - 1)
        d2 = jnp.where(in2, n2 + s2 - 1, _BPW + _L - 1)
        plsc.store_scatter(g0c, [d0], l0 - q0 * _R0)
        plsc.store_scatter(p0c, [d0], pos)
        plsc.store_scatter(c0c, [d0], q0 * 64)
        plsc.store_scatter(g1c, [d1], l1 - q1 * _R1)
        plsc.store_scatter(p1c, [d1], pos)
        plsc.store_scatter(c1c, [d1], q1 * 32)
        plsc.store_scatter(g2c, [d2], l2 - q2 * _R2)
        plsc.store_scatter(p2c, [d2], pos)
        plsc.store_scatter(c2c, [d2], q2 * 16)
        n0 = n0 + jnp.sum(in0.astype(jnp.int32))
        n1 = n1 + jnp.sum(in1.astype(jnp.int32))
        n2 = n2 + jnp.sum(in2.astype(jnp.int32))

    bufs = (bufa, bufb)
    sems = (sem_a, sem_b)

    def extract(pos_ref, cof_ref, buf, j, words):
        def chunk(m, _):
            pvec = pos_ref[pl.ds(j * _ISZ + m * _L, _L)]
            cvec = cof_ref[pl.ds(j * _ISZ + m * _L, _L)]
            for l in range(_L):
                s = pvec[l]
                cb = cvec[l]
                er = s >> 1
                ec = (s & 1) * 64
                for q in range(words):
                    ep_all[er, pl.ds(ec + q * _L, _L)] = \
                        buf[m * _L + l, pl.ds(cb + q * _L, _L)]
            return 0
        lax.fori_loop(0, _ISZ // _L, chunk, 0)

    for tbl, gc, pc, cc, n_t, words in (
            (t0, g0c, p0c, c0c, n0, 4),
            (t1, g1c, p1c, c1c, n1, 2),
            (t2, g2c, p2c, c2c, n2, 1)):
        for j in range(_NSTEP):
            @pl.when(n_t > j * _ISZ)
            def _fire(tbl=tbl, gc=gc, j=j):
                pltpu.async_copy(tbl.at[gc.at[pl.ds(j * _ISZ, _ISZ)]],
                                 bufs[j % 2], sems[j % 2])
            if j >= 1:
                @pl.when(n_t > (j - 1) * _ISZ)
                def _drain(tbl=tbl, gc=gc, pc=pc, cc=cc, j=j, words=words):
                    pltpu.make_async_copy(
                        tbl.at[gc.at[pl.ds((j - 1) * _ISZ, _ISZ)]],
                        bufs[(j - 1) % 2], sems[(j - 1) % 2]).wait()
                    extract(pc, cc, bufs[(j - 1) % 2], j - 1, words)
        @pl.when(n_t > (_NSTEP - 1) * _ISZ)
        def _drain_last(tbl=tbl, gc=gc, pc=pc, cc=cc, words=words):
            pltpu.make_async_copy(
                tbl.at[gc.at[pl.ds((_NSTEP - 1) * _ISZ, _ISZ)]],
                bufs[(_NSTEP - 1) % 2], sems[(_NSTEP - 1) % 2]).wait()
            extract(pc, cc, bufs[(_NSTEP - 1) % 2], _NSTEP - 1, words)

    pltpu.sync_copy(ep_all.at[pl.ds(0, _BPW // 2)],
                    ep_hbm.at[pl.ds(wid * (_BPW // 2), _BPW // 2)])


def _sc_gather(ids, t0v, t1v, t2v):
    mesh = plsc.VectorSubcoreMesh(core_axis_name="c", subcore_axis_name="s")
    f = functools.partial(
        pl.kernel,
        mesh=mesh,
        out_type=jax.ShapeDtypeStruct((_B // 2, 128), jnp.float32),
        scratch_types=[
            pltpu.VMEM((_BPW,), jnp.int32),
            pltpu.VMEM((_BPW + _L,), jnp.int32),
            pltpu.VMEM((_BPW + _L,), jnp.int32),
            pltpu.VMEM((_BPW + _L,), jnp.int32),
            pltpu.VMEM((_BPW + _L,), jnp.int32),
            pltpu.VMEM((_BPW + _L,), jnp.int32),
            pltpu.VMEM((_BPW + _L,), jnp.int32),
            pltpu.VMEM((_BPW + _L,), jnp.int32),
            pltpu.VMEM((_BPW + _L,), jnp.int32),
            pltpu.VMEM((_BPW + _L,), jnp.int32),
            pltpu.VMEM((_ISZ, 128), jnp.float32),
            pltpu.VMEM((_ISZ, 128), jnp.float32),
            pltpu.VMEM((_BPW // 2 + 1, 128), jnp.float32),
            pltpu.SemaphoreType.DMA,
            pltpu.SemaphoreType.DMA,
        ],
        compiler_params=pltpu.CompilerParams(needs_layout_passes=False),
    )(_sc_body)
    return f(ids, t0v, t1v, t2v)


def _combine_body(ids_ref, ep_ref, W1_ref, b1_ref, W2_ref, b2_ref, out_ref):
    x = ep_ref[...]
    ids = ids_ref[...]
    m0 = ids < _OFF1
    m1 = ids < _OFF2
    p1 = jnp.dot(x[:, :32], W1_ref[...],
                 preferred_element_type=jnp.float32) + b1_ref[...]
    p2 = jnp.dot(x[:, :16], W2_ref[...],
                 preferred_element_type=jnp.float32) + b2_ref[...]
    out_ref[...] = jnp.where(m0, x, jnp.where(m1, p1, p2))


def _tc_combine(ids, ep, W1, b1, W2, b2):
    rb = 1024
    grid = (_B // rb,)
    return pl.pallas_call(
        _combine_body,
        grid=grid,
        in_specs=[
            pl.BlockSpec((rb, 1), lambda i: (i, 0)),
            pl.BlockSpec((rb, 64), lambda i: (i, 0)),
            pl.BlockSpec((32, 64), lambda i: (0, 0)),
            pl.BlockSpec((1, 64), lambda i: (0, 0)),
            pl.BlockSpec((16, 64), lambda i: (0, 0)),
            pl.BlockSpec((1, 64), lambda i: (0, 0)),
        ],
        out_specs=pl.BlockSpec((rb, 64), lambda i: (i, 0)),
        out_shape=jax.ShapeDtypeStruct((_B, _BASE_DIM), jnp.float32),
    )(ids.reshape(_B, 1), ep, W1, b1.reshape(1, _BASE_DIM),
      W2, b2.reshape(1, _BASE_DIM))


def _band_body(*refs):
    ins = refs[:-1]
    out_ref = refs[-1]
    out_ref[...] = jnp.concatenate([r[...] for r in ins], axis=1)


def _band_view(emb, rows, width):
    nb = 128 // width
    rb = 512
    grid = (rows // rb,)
    nblk = rows // rb
    last = (emb.shape[0] + rb - 1) // rb - 1
    specs = [pl.BlockSpec((rb, width), functools.partial(
        lambda q, i: (jnp.minimum(q * nblk + i, last), 0), q))
        for q in range(nb)]
    return pl.pallas_call(
        _band_body,
        grid=grid,
        in_specs=specs,
        out_specs=pl.BlockSpec((rb, 128), lambda i: (i, 0)),
        out_shape=jax.ShapeDtypeStruct((rows, 128), jnp.float32),
    )(*([emb] * nb))


def kernel(inputs, emb0, emb1, emb2, W1, b1, W2, b2):
    t0v = _band_view(emb0, _R0, 64)
    t1v = _band_view(emb1, _R1, 32)
    t2v = _band_view(emb2, _R2, 16)
    ep = _sc_gather(inputs, t0v, t1v, t2v)
    return _tc_combine(inputs, ep.reshape(_B, _BASE_DIM), W1, b1, W2, b2)


# speedup vs baseline: 1.2459x; 1.2459x over previous
import functools

import jax
import jax.numpy as jnp
from jax import lax
from jax.experimental import pallas as pl
from jax.experimental.pallas import tpu as pltpu
from jax.experimental.pallas import tpu_sc as plsc

_OFF1 = 50000
_OFF2 = 80000
_BASE_DIM = 64
_B = 16384

_NC = 2
_NS = 16
_L = 16
_NW = _NC * _NS
_BPW = _B // _NW
_ISZ = 128
_NSTEP = _BPW // _ISZ

_R0 = 25088
_R1 = 7680
_R2 = 2560


def _sc_body(ids_hbm, t0, t1, t2, ep_hbm,
             ids_v, g0c, g1c, g2c, p0c, p1c, p2c,
             b0a, b0b, b1a, b1b, b2a, b2b, ep_all, sem_a, sem_b):
    sid = lax.axis_index("s")
    wid = sid * _NC + lax.axis_index("c")
    base = wid * _BPW
    pltpu.sync_copy(ids_hbm.at[pl.ds(base, _BPW)], ids_v)
    lane = lax.iota(jnp.int32, _L)
    trash = jnp.full((_L,), _BPW, jnp.int32)
    for c in range(_BPW // _L):
        pre = lane + c * _L
        sl = pl.ds(c * _L, _L)
        g0c[sl] = pre
        g1c[sl] = pre
        g2c[sl] = pre
        p0c[sl] = trash
        p1c[sl] = trash
        p2c[sl] = trash
    n0 = 0
    n1 = 0
    n2 = 0
    for c in range(_BPW // _L):
        v = ids_v[pl.ds(c * _L, _L)]
        pos = lane + c * _L
        in0 = v < _OFF1
        in1 = (v >= _OFF1) & (v < _OFF2)
        in2 = v >= _OFF2
        s0 = plsc.cumsum(in0.astype(jnp.int32))
        s1 = plsc.cumsum(in1.astype(jnp.int32))
        s2 = plsc.cumsum(in2.astype(jnp.int32))
        d0 = jnp.where(in0, n0 + s0 - 1, _BPW + _L - 1)
        d1 = jnp.where(in1, n1 + s1 - 1, _BPW + _L - 1)
        d2 = jnp.where(in2, n2 + s2 - 1, _BPW + _L - 1)
        plsc.store_scatter(g0c, [d0], v + 1)
        plsc.store_scatter(p0c, [d0], pos)
        plsc.store_scatter(g1c, [d1], v - (_OFF1 - 1))
        plsc.store_scatter(p1c, [d1], pos)
        plsc.store_scatter(g2c, [d2], v - (_OFF2 - 1))
        plsc.store_scatter(p2c, [d2], pos)
        n0 = n0 + jnp.sum(in0.astype(jnp.int32))
        n1 = n1 + jnp.sum(in1.astype(jnp.int32))
        n2 = n2 + jnp.sum(in2.astype(jnp.int32))

    def extract(pos_ref, buf, j, words):
        def chunk(m, _):
            pvec = pos_ref[pl.ds(j * _ISZ + m * _L, _L)]
            for l in range(_L):
                s = pvec[l]
                er = s >> 1
                ec = (s & 1) * 64
                for q in range(words):
                    ep_all[er, pl.ds(ec + q * _L, _L)] = \
                        buf[m * _L + l, pl.ds(q * _L, _L)]
            return 0
        lax.fori_loop(0, _ISZ // _L, chunk, 0)

    for tbl, gc, pc, n_t, words, bufs in (
            (t0, g0c, p0c, n0, 4, (b0a, b0b)),
            (t1, g1c, p1c, n1, 2, (b1a, b1b)),
            (t2, g2c, p2c, n2, 1, (b2a, b2b))):
        sems = (sem_a, sem_b)
        for j in range(_NSTEP):
            @pl.when(n_t > j * _ISZ)
            def _fire(tbl=tbl, gc=gc, j=j, bufs=bufs):
                pltpu.async_copy(tbl.at[gc.at[pl.ds(j * _ISZ, _ISZ)]],
                                 bufs[j % 2], sems[j % 2])
            if j >= 1:
                @pl.when(n_t > (j - 1) * _ISZ)
                def _drain(tbl=tbl, gc=gc, pc=pc, j=j, words=words,
                           bufs=bufs):
                    pltpu.make_async_copy(
                        tbl.at[gc.at[pl.ds((j - 1) * _ISZ, _ISZ)]],
                        bufs[(j - 1) % 2], sems[(j - 1) % 2]).wait()
                    extract(pc, bufs[(j - 1) % 2], j - 1, words)
        @pl.when(n_t > (_NSTEP - 1) * _ISZ)
        def _drain_last(tbl=tbl, gc=gc, pc=pc, words=words, bufs=bufs):
            pltpu.make_async_copy(
                tbl.at[gc.at[pl.ds((_NSTEP - 1) * _ISZ, _ISZ)]],
                bufs[(_NSTEP - 1) % 2], sems[(_NSTEP - 1) % 2]).wait()
            extract(pc, bufs[(_NSTEP - 1) % 2], _NSTEP - 1, words)

    pltpu.sync_copy(ep_all.at[pl.ds(0, _BPW // 2)],
                    ep_hbm.at[pl.ds(wid * (_BPW // 2), _BPW // 2)])


def _sc_gather(ids, t0v, t1v, t2v):
    mesh = plsc.VectorSubcoreMesh(core_axis_name="c", subcore_axis_name="s")
    f = functools.partial(
        pl.kernel,
        mesh=mesh,
        out_type=jax.ShapeDtypeStruct((_B // 2, 128), jnp.float32),
        scratch_types=[
            pltpu.VMEM((_BPW,), jnp.int32),
            pltpu.VMEM((_BPW + _L,), jnp.int32),
            pltpu.VMEM((_BPW + _L,), jnp.int32),
            pltpu.VMEM((_BPW + _L,), jnp.int32),
            pltpu.VMEM((_BPW + _L,), jnp.int32),
            pltpu.VMEM((_BPW + _L,), jnp.int32),
            pltpu.VMEM((_BPW + _L,), jnp.int32),
            pltpu.VMEM((_ISZ, 64), jnp.float32),
            pltpu.VMEM((_ISZ, 64), jnp.float32),
            pltpu.VMEM((_ISZ, 32), jnp.float32),
            pltpu.VMEM((_ISZ, 32), jnp.float32),
            pltpu.VMEM((_ISZ, 16), jnp.float32),
            pltpu.VMEM((_ISZ, 16), jnp.float32),
            pltpu.VMEM((_BPW // 2 + 1, 128), jnp.float32),
            pltpu.SemaphoreType.DMA,
            pltpu.SemaphoreType.DMA,
        ],
        compiler_params=pltpu.CompilerParams(
            use_tc_tiling_on_sc=False, needs_layout_passes=False),
    )(_sc_body)
    return f(ids, t0v, t1v, t2v)


def _combine_body(ids_ref, ep_ref, W1_ref, b1_ref, W2_ref, b2_ref, out_ref):
    x = ep_ref[...]
    ids = ids_ref[...]
    m0 = ids < _OFF1
    m1 = ids < _OFF2
    p1 = jnp.dot(x[:, :32], W1_ref[...],
                 preferred_element_type=jnp.float32) + b1_ref[...]
    p2 = jnp.dot(x[:, :16], W2_ref[...],
                 preferred_element_type=jnp.float32) + b2_ref[...]
    out_ref[...] = jnp.where(m0, x, jnp.where(m1, p1, p2))


def _tc_combine(ids, ep, W1, b1, W2, b2):
    rb = 1024
    grid = (_B // rb,)
    return pl.pallas_call(
        _combine_body,
        grid=grid,
        in_specs=[
            pl.BlockSpec((rb, 1), lambda i: (i, 0)),
            pl.BlockSpec((rb, 64), lambda i: (i, 0)),
            pl.BlockSpec((32, 64), lambda i: (0, 0)),
            pl.BlockSpec((1, 64), lambda i: (0, 0)),
            pl.BlockSpec((16, 64), lambda i: (0, 0)),
            pl.BlockSpec((1, 64), lambda i: (0, 0)),
        ],
        out_specs=pl.BlockSpec((rb, 64), lambda i: (i, 0)),
        out_shape=jax.ShapeDtypeStruct((_B, _BASE_DIM), jnp.float32),
    )(ids.reshape(_B, 1), ep, W1, b1.reshape(1, _BASE_DIM),
      W2, b2.reshape(1, _BASE_DIM))


def kernel(inputs, emb0, emb1, emb2, W1, b1, W2, b2):
    ep = _sc_gather(inputs, emb0, emb1, emb2)
    return _tc_combine(inputs, ep.reshape(_B, _BASE_DIM), W1, b1, W2, b2)


# speedup vs baseline: 1.3072x; 1.0492x over previous
import functools

import jax
import jax.numpy as jnp
from jax import lax
from jax.experimental import pallas as pl
from jax.experimental.pallas import tpu as pltpu
from jax.experimental.pallas import tpu_sc as plsc

_OFF1 = 50000
_OFF2 = 80000
_BASE_DIM = 64
_B = 16384

_NC = 2
_NS = 16
_L = 16
_NW = _NC * _NS
_BPW = _B // _NW
_ISZ = 128
_NSTEP = _BPW // _ISZ

_R0 = 25088
_R1 = 7680
_R2 = 2560


def _sc_body(ids_hbm, t0, t1, t2, ep_hbm,
             ids_v, g0c, g1c, g2c, p0c, p1c, p2c,
             b0a, b0b, b1a, b1b, b2a, b2b, ep_all, sem_a, sem_b):
    sid = lax.axis_index("s")
    wid = sid * _NC + lax.axis_index("c")
    base = wid * _BPW
    pltpu.sync_copy(ids_hbm.at[pl.ds(base, _BPW)], ids_v)
    lane = lax.iota(jnp.int32, _L)
    trash = jnp.full((_L,), _BPW, jnp.int32)
    for c in range(_BPW // _L):
        pre = lane + c * _L
        sl = pl.ds(c * _L, _L)
        g0c[sl] = pre
        g1c[sl] = pre
        g2c[sl] = pre
        p0c[sl] = trash
        p1c[sl] = trash
        p2c[sl] = trash
    n0 = 0
    n1 = 0
    n2 = 0
    for c in range(_BPW // _L):
        v = ids_v[pl.ds(c * _L, _L)]
        pos = lane + c * _L
        in0 = v < _OFF1
        in1 = (v >= _OFF1) & (v < _OFF2)
        in2 = v >= _OFF2
        s0 = plsc.cumsum(in0.astype(jnp.int32))
        s1 = plsc.cumsum(in1.astype(jnp.int32))
        s2 = plsc.cumsum(in2.astype(jnp.int32))
        d0 = jnp.where(in0, n0 + s0 - 1, _BPW + _L - 1)
        d1 = jnp.where(in1, n1 + s1 - 1, _BPW + _L - 1)
        d2 = jnp.where(in2, n2 + s2 - 1, _BPW + _L - 1)
        plsc.store_scatter(g0c, [d0], v + 1)
        plsc.store_scatter(p0c, [d0], pos)
        plsc.store_scatter(g1c, [d1], v - (_OFF1 - 1))
        plsc.store_scatter(p1c, [d1], pos)
        plsc.store_scatter(g2c, [d2], v - (_OFF2 - 1))
        plsc.store_scatter(p2c, [d2], pos)
        n0 = n0 + jnp.sum(in0.astype(jnp.int32))
        n1 = n1 + jnp.sum(in1.astype(jnp.int32))
        n2 = n2 + jnp.sum(in2.astype(jnp.int32))

    def extract(pos_ref, buf, j, words):
        def chunk(m, _):
            pvec = pos_ref[pl.ds(j * _ISZ + m * _L, _L)]
            for l in range(_L):
                s = pvec[l]
                er = s >> 1
                ec = (s & 1) * 64
                for q in range(words):
                    ep_all[er, pl.ds(ec + q * _L, _L)] = \
                        buf[m * _L + l, pl.ds(q * _L, _L)]
            return 0
        lax.fori_loop(0, _ISZ // _L, chunk, 0)

    for tbl, gc, pc, n_t, words, bufs in (
            (t0, g0c, p0c, n0, 4, (b0a, b0b)),
            (t1, g1c, p1c, n1, 2, (b1a, b1b)),
            (t2, g2c, p2c, n2, 1, (b2a, b2b))):
        sems = (sem_a, sem_b)
        for j in range(_NSTEP):
            @pl.when(n_t > j * _ISZ)
            def _fire(tbl=tbl, gc=gc, j=j, bufs=bufs):
                pltpu.async_copy(tbl.at[gc.at[pl.ds(j * _ISZ, _ISZ)]],
                                 bufs[j % 2], sems[j % 2])
            if j >= 1:
                @pl.when(n_t > (j - 1) * _ISZ)
                def _drain(tbl=tbl, gc=gc, pc=pc, j=j, words=words,
                           bufs=bufs):
                    pltpu.make_async_copy(
                        tbl.at[gc.at[pl.ds((j - 1) * _ISZ, _ISZ)]],
                        bufs[(j - 1) % 2], sems[(j - 1) % 2]).wait()
                    extract(pc, bufs[(j - 1) % 2], j - 1, words)
        @pl.when(n_t > (_NSTEP - 1) * _ISZ)
        def _drain_last(tbl=tbl, gc=gc, pc=pc, words=words, bufs=bufs):
            pltpu.make_async_copy(
                tbl.at[gc.at[pl.ds((_NSTEP - 1) * _ISZ, _ISZ)]],
                bufs[(_NSTEP - 1) % 2], sems[(_NSTEP - 1) % 2]).wait()
            extract(pc, bufs[(_NSTEP - 1) % 2], _NSTEP - 1, words)

    pltpu.sync_copy(ep_all.at[pl.ds(0, _BPW // 2)],
                    ep_hbm.at[pl.ds(wid * (_BPW // 2), _BPW // 2)])


def _sc_gather(ids, t0v, t1v, t2v):
    mesh = plsc.VectorSubcoreMesh(core_axis_name="c", subcore_axis_name="s")
    f = functools.partial(
        pl.kernel,
        mesh=mesh,
        out_type=jax.ShapeDtypeStruct((_B // 2, 128), jnp.float32),
        scratch_types=[
            pltpu.VMEM((_BPW,), jnp.int32),
            pltpu.VMEM((_BPW + _L,), jnp.int32),
            pltpu.VMEM((_BPW + _L,), jnp.int32),
            pltpu.VMEM((_BPW + _L,), jnp.int32),
            pltpu.VMEM((_BPW + _L,), jnp.int32),
            pltpu.VMEM((_BPW + _L,), jnp.int32),
            pltpu.VMEM((_BPW + _L,), jnp.int32),
            pltpu.VMEM((_ISZ, 64), jnp.float32),
            pltpu.VMEM((_ISZ, 64), jnp.float32),
            pltpu.VMEM((_ISZ, 32), jnp.float32),
            pltpu.VMEM((_ISZ, 32), jnp.float32),
            pltpu.VMEM((_ISZ, 16), jnp.float32),
            pltpu.VMEM((_ISZ, 16), jnp.float32),
            pltpu.VMEM((_BPW // 2 + 1, 128), jnp.float32),
            pltpu.SemaphoreType.DMA,
            pltpu.SemaphoreType.DMA,
        ],
        compiler_params=pltpu.CompilerParams(
            use_tc_tiling_on_sc=False, needs_layout_passes=False),
    )(_sc_body)
    return f(ids, t0v, t1v, t2v)


def _combine_body(ids_ref, ep_ref, W1_ref, b1_ref, W2_ref, b2_ref, out_ref):
    x = ep_ref[...]
    ids = ids_ref[...]
    m0 = ids < _OFF1
    m1 = ids < _OFF2
    p1 = jnp.dot(x[:, :32], W1_ref[...],
                 preferred_element_type=jnp.float32) + b1_ref[...]
    p2 = jnp.dot(x[:, :16], W2_ref[...],
                 preferred_element_type=jnp.float32) + b2_ref[...]
    out_ref[...] = jnp.where(m0, x, jnp.where(m1, p1, p2))


def _tc_combine(ids, ep, W1, b1, W2, b2):
    rb = 4096
    grid = (_B // rb,)
    return pl.pallas_call(
        _combine_body,
        grid=grid,
        in_specs=[
            pl.BlockSpec((rb, 1), lambda i: (i, 0)),
            pl.BlockSpec((rb, 64), lambda i: (i, 0)),
            pl.BlockSpec((32, 64), lambda i: (0, 0)),
            pl.BlockSpec((1, 64), lambda i: (0, 0)),
            pl.BlockSpec((16, 64), lambda i: (0, 0)),
            pl.BlockSpec((1, 64), lambda i: (0, 0)),
        ],
        out_specs=pl.BlockSpec((rb, 64), lambda i: (i, 0)),
        out_shape=jax.ShapeDtypeStruct((_B, _BASE_DIM), jnp.float32),
    )(ids.reshape(_B, 1), ep, W1, b1.reshape(1, _BASE_DIM),
      W2, b2.reshape(1, _BASE_DIM))


def kernel(inputs, emb0, emb1, emb2, W1, b1, W2, b2):
    ep = _sc_gather(inputs, emb0, emb1, emb2)
    return _tc_combine(inputs, ep.reshape(_B, _BASE_DIM), W1, b1, W2, b2)


# speedup vs baseline: 1.3147x; 1.0058x over previous
import functools

import jax
import jax.numpy as jnp
from jax import lax
from jax.experimental import pallas as pl
from jax.experimental.pallas import tpu as pltpu
from jax.experimental.pallas import tpu_sc as plsc

_OFF1 = 50000
_OFF2 = 80000
_BASE_DIM = 64
_B = 16384

_NC = 2
_NS = 16
_L = 16
_NW = _NC * _NS
_BPW = _B // _NW
_ISZ = 128
_NSTEP = _BPW // _ISZ

_R0 = 25088
_R1 = 7680
_R2 = 2560


def _sc_body(ids_hbm, t0, t1, t2, ep_hbm,
             ids_v, g0c, g1c, g2c, p0c, p1c, p2c,
             b0a, b0b, b1a, b1b, b2a, b2b, ep_all, sem_a, sem_b):
    sid = lax.axis_index("s")
    wid = sid * _NC + lax.axis_index("c")
    base = wid * _BPW
    pltpu.sync_copy(ids_hbm.at[pl.ds(base, _BPW)], ids_v)
    lane = lax.iota(jnp.int32, _L)
    trash = jnp.full((_L,), _BPW, jnp.int32)
    for c in range(_BPW // _L):
        pre = lane + c * _L
        sl = pl.ds(c * _L, _L)
        g0c[sl] = pre
        g1c[sl] = pre
        g2c[sl] = pre
        p0c[sl] = trash
        p1c[sl] = trash
        p2c[sl] = trash
    n0 = 0
    n1 = 0
    n2 = 0
    for c in range(_BPW // _L):
        v = ids_v[pl.ds(c * _L, _L)]
        pos = lane + c * _L
        in0 = v < _OFF1
        in1 = (v >= _OFF1) & (v < _OFF2)
        in2 = v >= _OFF2
        s0 = plsc.cumsum(in0.astype(jnp.int32))
        s1 = plsc.cumsum(in1.astype(jnp.int32))
        s2 = plsc.cumsum(in2.astype(jnp.int32))
        d0 = jnp.where(in0, n0 + s0 - 1, _BPW + _L - 1)
        d1 = jnp.where(in1, n1 + s1 - 1, _BPW + _L - 1)
        d2 = jnp.where(in2, n2 + s2 - 1, _BPW + _L - 1)
        plsc.store_scatter(g0c, [d0], v + 1)
        plsc.store_scatter(p0c, [d0], pos)
        plsc.store_scatter(g1c, [d1], v - (_OFF1 - 1))
        plsc.store_scatter(p1c, [d1], pos)
        plsc.store_scatter(g2c, [d2], v - (_OFF2 - 1))
        plsc.store_scatter(p2c, [d2], pos)
        n0 = n0 + jnp.sum(in0.astype(jnp.int32))
        n1 = n1 + jnp.sum(in1.astype(jnp.int32))
        n2 = n2 + jnp.sum(in2.astype(jnp.int32))

    def extract(pos_ref, buf, j, words):
        def chunk(m, _):
            pvec = pos_ref[pl.ds(j * _ISZ + m * _L, _L)]
            for l in range(_L):
                s = pvec[l]
                er = s >> 1
                ec = (s & 1) * 64
                for q in range(words):
                    ep_all[er, pl.ds(ec + q * _L, _L)] = \
                        buf[m * _L + l, pl.ds(q * _L, _L)]
            return 0
        lax.fori_loop(0, _ISZ // _L, chunk, 0)

    for tbl, gc, pc, n_t, words, bufs in (
            (t0, g0c, p0c, n0, 4, (b0a, b0b)),
            (t1, g1c, p1c, n1, 2, (b1a, b1b)),
            (t2, g2c, p2c, n2, 1, (b2a, b2b))):
        sems = (sem_a, sem_b)
        for j in range(_NSTEP):
            @pl.when(n_t > j * _ISZ)
            def _fire(tbl=tbl, gc=gc, j=j, bufs=bufs):
                pltpu.async_copy(tbl.at[gc.at[pl.ds(j * _ISZ, _ISZ)]],
                                 bufs[j % 2], sems[j % 2])
            if j >= 1:
                @pl.when(n_t > (j - 1) * _ISZ)
                def _drain(tbl=tbl, gc=gc, pc=pc, j=j, words=words,
                           bufs=bufs):
                    pltpu.make_async_copy(
                        tbl.at[gc.at[pl.ds((j - 1) * _ISZ, _ISZ)]],
                        bufs[(j - 1) % 2], sems[(j - 1) % 2]).wait()
                    extract(pc, bufs[(j - 1) % 2], j - 1, words)
        @pl.when(n_t > (_NSTEP - 1) * _ISZ)
        def _drain_last(tbl=tbl, gc=gc, pc=pc, words=words, bufs=bufs):
            pltpu.make_async_copy(
                tbl.at[gc.at[pl.ds((_NSTEP - 1) * _ISZ, _ISZ)]],
                bufs[(_NSTEP - 1) % 2], sems[(_NSTEP - 1) % 2]).wait()
            extract(pc, bufs[(_NSTEP - 1) % 2], _NSTEP - 1, words)

    pltpu.sync_copy(ep_all.at[pl.ds(0, _BPW // 2)],
                    ep_hbm.at[pl.ds(wid * (_BPW // 2), _BPW // 2)])


def _sc_gather(ids, t0v, t1v, t2v):
    mesh = plsc.VectorSubcoreMesh(core_axis_name="c", subcore_axis_name="s")
    f = functools.partial(
        pl.kernel,
        mesh=mesh,
        out_type=jax.ShapeDtypeStruct((_B // 2, 128), jnp.float32),
        scratch_types=[
            pltpu.VMEM((_BPW,), jnp.int32),
            pltpu.VMEM((_BPW + _L,), jnp.int32),
            pltpu.VMEM((_BPW + _L,), jnp.int32),
            pltpu.VMEM((_BPW + _L,), jnp.int32),
            pltpu.VMEM((_BPW + _L,), jnp.int32),
            pltpu.VMEM((_BPW + _L,), jnp.int32),
            pltpu.VMEM((_BPW + _L,), jnp.int32),
            pltpu.VMEM((_ISZ, 64), jnp.float32),
            pltpu.VMEM((_ISZ, 64), jnp.float32),
            pltpu.VMEM((_ISZ, 32), jnp.float32),
            pltpu.VMEM((_ISZ, 32), jnp.float32),
            pltpu.VMEM((_ISZ, 16), jnp.float32),
            pltpu.VMEM((_ISZ, 16), jnp.float32),
            pltpu.VMEM((_BPW // 2 + 1, 128), jnp.float32),
            pltpu.SemaphoreType.DMA,
            pltpu.SemaphoreType.DMA,
        ],
        compiler_params=pltpu.CompilerParams(
            use_tc_tiling_on_sc=False, needs_layout_passes=False),
    )(_sc_body)
    return f(ids, t0v, t1v, t2v)


def _combine_body(ids_ref, ep_ref, W1_ref, b1_ref, W2_ref, b2_ref, out_ref):
    x = ep_ref[...]
    ids = ids_ref[...]
    m0 = ids < _OFF1
    m1 = ids < _OFF2
    p1 = jnp.dot(x[:, :32], W1_ref[...],
                 preferred_element_type=jnp.float32) + b1_ref[...]
    p2 = jnp.dot(x[:, :16], W2_ref[...],
                 preferred_element_type=jnp.float32) + b2_ref[...]
    out_ref[...] = jnp.where(m0, x, jnp.where(m1, p1, p2))


def _tc_combine(ids, ep, W1, b1, W2, b2):
    rb = 8192
    grid = (_B // rb,)
    return pl.pallas_call(
        _combine_body,
        grid=grid,
        in_specs=[
            pl.BlockSpec((rb, 1), lambda i: (i, 0)),
            pl.BlockSpec((rb, 64), lambda i: (i, 0)),
            pl.BlockSpec((32, 64), lambda i: (0, 0)),
            pl.BlockSpec((1, 64), lambda i: (0, 0)),
            pl.BlockSpec((16, 64), lambda i: (0, 0)),
            pl.BlockSpec((1, 64), lambda i: (0, 0)),
        ],
        out_specs=pl.BlockSpec((rb, 64), lambda i: (i, 0)),
        out_shape=jax.ShapeDtypeStruct((_B, _BASE_DIM), jnp.float32),
    )(ids.reshape(_B, 1), ep, W1, b1.reshape(1, _BASE_DIM),
      W2, b2.reshape(1, _BASE_DIM))


def kernel(inputs, emb0, emb1, emb2, W1, b1, W2, b2):
    ep = _sc_gather(inputs, emb0, emb1, emb2)
    return _tc_combine(inputs, ep.reshape(_B, _BASE_DIM), W1, b1, W2, b2)


# speedup vs baseline: 1.3166x; 1.0014x over previous
import functools

import jax
import jax.numpy as jnp
from jax import lax
from jax.experimental import pallas as pl
from jax.experimental.pallas import tpu as pltpu
from jax.experimental.pallas import tpu_sc as plsc

_OFF1 = 50000
_OFF2 = 80000
_BASE_DIM = 64
_B = 16384

_NC = 2
_NS = 16
_L = 16
_NW = _NC * _NS
_BPW = _B // _NW
_ISZ = 128
_NSTEP = _BPW // _ISZ

_R0 = 25088
_R1 = 7680
_R2 = 2560


def _sc_body(ids_hbm, t0, t1, t2, ep_hbm,
             ids_v, g0c, g1c, g2c, p0c, p1c, p2c,
             b0a, b0b, b1a, b1b, b2a, b2b, ep_all, sem_a, sem_b):
    sid = lax.axis_index("s")
    wid = sid * _NC + lax.axis_index("c")
    base = wid * _BPW
    pltpu.sync_copy(ids_hbm.at[pl.ds(base, _BPW)], ids_v)
    lane = lax.iota(jnp.int32, _L)
    trash = jnp.full((_L,), _BPW, jnp.int32)
    for c in range(_BPW // _L):
        pre = lane + c * _L
        sl = pl.ds(c * _L, _L)
        g0c[sl] = pre
        g1c[sl] = pre
        g2c[sl] = pre
        p0c[sl] = trash
        p1c[sl] = trash
        p2c[sl] = trash
    n0 = 0
    n1 = 0
    n2 = 0
    for c in range(_BPW // _L):
        v = ids_v[pl.ds(c * _L, _L)]
        pos = lane + c * _L
        in0 = v < _OFF1
        in1 = (v >= _OFF1) & (v < _OFF2)
        in2 = v >= _OFF2
        s0 = plsc.cumsum(in0.astype(jnp.int32))
        s1 = plsc.cumsum(in1.astype(jnp.int32))
        s2 = plsc.cumsum(in2.astype(jnp.int32))
        d0 = jnp.where(in0, n0 + s0 - 1, _BPW + _L - 1)
        d1 = jnp.where(in1, n1 + s1 - 1, _BPW + _L - 1)
        d2 = jnp.where(in2, n2 + s2 - 1, _BPW + _L - 1)
        plsc.store_scatter(g0c, [d0], v + 1)
        plsc.store_scatter(p0c, [d0], pos)
        plsc.store_scatter(g1c, [d1], v - (_OFF1 - 1))
        plsc.store_scatter(p1c, [d1], pos)
        plsc.store_scatter(g2c, [d2], v - (_OFF2 - 1))
        plsc.store_scatter(p2c, [d2], pos)
        n0 = n0 + jnp.sum(in0.astype(jnp.int32))
        n1 = n1 + jnp.sum(in1.astype(jnp.int32))
        n2 = n2 + jnp.sum(in2.astype(jnp.int32))

    def extract(pos_ref, buf, j, words):
        def chunk(m, _):
            pvec = pos_ref[pl.ds(j * _ISZ + m * _L, _L)]
            for l in range(_L):
                s = pvec[l]
                for q in range(words):
                    ep_all[s, pl.ds(q * _L, _L)] = \
                        buf[m * _L + l, pl.ds(q * _L, _L)]
            return 0
        lax.fori_loop(0, _ISZ // _L, chunk, 0)

    for tbl, gc, pc, n_t, words, bufs in (
            (t0, g0c, p0c, n0, 4, (b0a, b0b)),
            (t1, g1c, p1c, n1, 2, (b1a, b1b)),
            (t2, g2c, p2c, n2, 1, (b2a, b2b))):
        sems = (sem_a, sem_b)
        for j in range(_NSTEP):
            @pl.when(n_t > j * _ISZ)
            def _fire(tbl=tbl, gc=gc, j=j, bufs=bufs):
                pltpu.async_copy(tbl.at[gc.at[pl.ds(j * _ISZ, _ISZ)]],
                                 bufs[j % 2], sems[j % 2])
            if j >= 1:
                @pl.when(n_t > (j - 1) * _ISZ)
                def _drain(tbl=tbl, gc=gc, pc=pc, j=j, words=words,
                           bufs=bufs):
                    pltpu.make_async_copy(
                        tbl.at[gc.at[pl.ds((j - 1) * _ISZ, _ISZ)]],
                        bufs[(j - 1) % 2], sems[(j - 1) % 2]).wait()
                    extract(pc, bufs[(j - 1) % 2], j - 1, words)
        @pl.when(n_t > (_NSTEP - 1) * _ISZ)
        def _drain_last(tbl=tbl, gc=gc, pc=pc, words=words, bufs=bufs):
            pltpu.make_async_copy(
                tbl.at[gc.at[pl.ds((_NSTEP - 1) * _ISZ, _ISZ)]],
                bufs[(_NSTEP - 1) % 2], sems[(_NSTEP - 1) % 2]).wait()
            extract(pc, bufs[(_NSTEP - 1) % 2], _NSTEP - 1, words)

    pltpu.sync_copy(ep_all.at[pl.ds(0, _BPW)],
                    ep_hbm.at[pl.ds(base, _BPW)])


def _sc_gather(ids, t0v, t1v, t2v):
    mesh = plsc.VectorSubcoreMesh(core_axis_name="c", subcore_axis_name="s")
    f = functools.partial(
        pl.kernel,
        mesh=mesh,
        out_type=jax.ShapeDtypeStruct((_B, _BASE_DIM), jnp.float32),
        scratch_types=[
            pltpu.VMEM((_BPW,), jnp.int32),
            pltpu.VMEM((_BPW + _L,), jnp.int32),
            pltpu.VMEM((_BPW + _L,), jnp.int32),
            pltpu.VMEM((_BPW + _L,), jnp.int32),
            pltpu.VMEM((_BPW + _L,), jnp.int32),
            pltpu.VMEM((_BPW + _L,), jnp.int32),
            pltpu.VMEM((_BPW + _L,), jnp.int32),
            pltpu.VMEM((_ISZ, 64), jnp.float32),
            pltpu.VMEM((_ISZ, 64), jnp.float32),
            pltpu.VMEM((_ISZ, 32), jnp.float32),
            pltpu.VMEM((_ISZ, 32), jnp.float32),
            pltpu.VMEM((_ISZ, 16), jnp.float32),
            pltpu.VMEM((_ISZ, 16), jnp.float32),
            pltpu.VMEM((_BPW + 1, _BASE_DIM), jnp.float32),
            pltpu.SemaphoreType.DMA,
            pltpu.SemaphoreType.DMA,
        ],
        compiler_params=pltpu.CompilerParams(
            use_tc_tiling_on_sc=False, needs_layout_passes=False),
    )(_sc_body)
    return f(ids, t0v, t1v, t2v)


def _combine_body(ids_ref, ep_ref, W1_ref, b1_ref, W2_ref, b2_ref, out_ref):
    x = ep_ref[...]
    ids = ids_ref[...]
    m0 = ids < _OFF1
    m1 = ids < _OFF2
    p1 = jnp.dot(x[:, :32], W1_ref[...],
                 preferred_element_type=jnp.float32) + b1_ref[...]
    p2 = jnp.dot(x[:, :16], W2_ref[...],
                 preferred_element_type=jnp.float32) + b2_ref[...]
    out_ref[...] = jnp.where(m0, x, jnp.where(m1, p1, p2))


def _tc_combine(ids, ep, W1, b1, W2, b2):
    rb = 8192
    grid = (_B // rb,)
    return pl.pallas_call(
        _combine_body,
        grid=grid,
        in_specs=[
            pl.BlockSpec((rb, 1), lambda i: (i, 0)),
            pl.BlockSpec((rb, 64), lambda i: (i, 0)),
            pl.BlockSpec((32, 64), lambda i: (0, 0)),
            pl.BlockSpec((1, 64), lambda i: (0, 0)),
            pl.BlockSpec((16, 64), lambda i: (0, 0)),
            pl.BlockSpec((1, 64), lambda i: (0, 0)),
        ],
        out_specs=pl.BlockSpec((rb, 64), lambda i: (i, 0)),
        out_shape=jax.ShapeDtypeStruct((_B, _BASE_DIM), jnp.float32),
    )(ids.reshape(_B, 1), ep, W1, b1.reshape(1, _BASE_DIM),
      W2, b2.reshape(1, _BASE_DIM))


def kernel(inputs, emb0, emb1, emb2, W1, b1, W2, b2):
    ep = _sc_gather(inputs, emb0, emb1, emb2)
    return _tc_combine(inputs, ep, W1, b1, W2, b2)
